# Initial kernel scaffold; baseline (speedup 1.0000x reference)
import jax
import jax.numpy as jnp
from jax.experimental import pallas as pl


def kernel(x, batch, edge_index, W1, b1, W2, b2, Wg, bg):
    raise NotImplementedError("write your pallas kernel here")


# speedup vs baseline: 63.6900x; 63.6900x over previous
import functools

import jax
import jax.numpy as jnp
from jax import lax
from jax.experimental import pallas as pl
from jax.experimental.pallas import tpu as pltpu
from jax.experimental.pallas import tpu_sc as plsc

NC, NS, LANES = 2, 16, 16
NW = NC * NS

N, C, E, G = 10000, 128, 320000, 64
BN = 1000
GRID = N // BN

COLS = 1024
ROWS = NS
NPAD = ROWS * COLS
PADNODE = NPAD - 1

ER_W = 80
EPAD = NW * ER_W * 128
ER_ALL = EPAD // 128
ER_TILE = ER_ALL // NS
FIRE = 16


def _stats_body(x_ref, bf_ref, w1_ref, b1_ref, w2_ref, b2_ref, wg_ref,
                h_ref, chn_ref, sums_ref, cnt_ref):
    i = pl.program_id(0)
    xb = x_ref[...]
    bf = bf_ref[...]
    iot = lax.broadcasted_iota(jnp.int32, (BN, G), 1).astype(jnp.float32)
    onehot = (bf == iot).astype(jnp.float32)
    psums = lax.dot_general(onehot, xb, (((0,), (0,)), ((), ())),
                            preferred_element_type=jnp.float32)
    pcnt = lax.dot_general(onehot, jnp.ones_like(xb), (((0,), (0,)), ((), ())),
                           preferred_element_type=jnp.float32)

    @pl.when(i == 0)
    def _():
        sums_ref[...] = jnp.zeros_like(sums_ref)
        cnt_ref[...] = jnp.zeros_like(cnt_ref)

    sums_ref[...] += psums
    cnt_ref[...] += pcnt
    h_ref[...] = jnp.dot(xb, wg_ref[...], preferred_element_type=jnp.float32)

    @pl.when(i == GRID - 1)
    def _():
        means = sums_ref[...] / jnp.maximum(cnt_ref[...], 1.0)
        t = jnp.maximum(
            jnp.dot(means, w1_ref[...], preferred_element_type=jnp.float32)
            + b1_ref[...], 0.0)
        chn_ref[...] = jax.nn.sigmoid(
            jnp.dot(t, w2_ref[...], preferred_element_type=jnp.float32)
            + b2_ref[...])


def _run_stats(x, bf, W1, b1, W2, b2, Wg):
    return pl.pallas_call(
        _stats_body,
        grid=(GRID,),
        in_specs=[
            pl.BlockSpec((BN, C), lambda i: (i, 0)),
            pl.BlockSpec((BN, 1), lambda i: (i, 0)),
            pl.BlockSpec((C, G), lambda i: (0, 0)),
            pl.BlockSpec((1, G), lambda i: (0, 0)),
            pl.BlockSpec((G, C), lambda i: (0, 0)),
            pl.BlockSpec((1, C), lambda i: (0, 0)),
            pl.BlockSpec((C, 1), lambda i: (0, 0)),
        ],
        out_specs=[
            pl.BlockSpec((BN, 1), lambda i: (i, 0)),
            pl.BlockSpec((G, C), lambda i: (0, 0)),
        ],
        out_shape=[
            jax.ShapeDtypeStruct((N, 1), jnp.float32),
            jax.ShapeDtypeStruct((G, C), jnp.float32),
        ],
        scratch_shapes=[
            pltpu.VMEM((G, C), jnp.float32),
            pltpu.VMEM((G, C), jnp.float32),
        ],
    )(x, bf, W1, b1, W2, b2, Wg)


def _rsqrt3(d):
    yi = plsc.bitcast(d, jnp.int32)
    yi = 0x5F3759DF - lax.shift_right_arithmetic(yi, 1)
    y = plsc.bitcast(yi, jnp.float32)
    half = 0.5 * d
    y = y * (1.5 - half * y * y)
    y = y * (1.5 - half * y * y)
    y = y * (1.5 - half * y * y)
    return y


def _fire_scatter_chunked(n_rows, vals_row, dst_rows, target, sem):
    for base in range(0, n_rows, FIRE):
        k = min(FIRE, n_rows - base)
        descs = [
            pltpu.async_copy(vals_row(base + j),
                             target.at[dst_rows(base + j)], sem, add=True)
            for j in range(k)
        ]
        for dsc in descs:
            dsc.wait()


def _sc_body(dst2d, src2d, h2d, accp, dinv_out,
             zeros_v, ones_v, dstd_v, src_v, dstw_v, qv_v, q_v, d_v, h_v,
             deg_s, q_s, acc_s, sem, sem2):
    c = lax.axis_index("c")
    s = lax.axis_index("s")
    wid = c * NS + s

    def zbody(j, _):
        zeros_v[pl.ds(j * 16, 16)] = jnp.zeros((16,), jnp.float32)
        return 0

    lax.fori_loop(0, COLS // 16, zbody, 0)
    for k in range(128 // 16):
        ones_v[pl.ds(k * 16, 16)] = jnp.ones((16,), jnp.float32)

    cp_dd = pltpu.async_copy(dst2d.at[pl.ds(s * ER_TILE, ER_TILE)], dstd_v, sem2)
    cp_sw = pltpu.async_copy(src2d.at[pl.ds(wid * ER_W, ER_W)], src_v, sem2)
    cp_dw = pltpu.async_copy(dst2d.at[pl.ds(wid * ER_W, ER_W)], dstw_v, sem2)
    cp_h = pltpu.async_copy(h2d.at[s], h_v, sem2)

    pltpu.sync_copy(zeros_v, deg_s.at[pl.ds(s * COLS, COLS)])
    pltpu.sync_copy(zeros_v, acc_s.at[pl.ds(s * COLS, COLS)])
    plsc.subcore_barrier()

    cp_dd.wait()
    _fire_scatter_chunked(ER_TILE, lambda j: ones_v,
                          lambda j: dstd_v.at[j], deg_s, sem)
    plsc.subcore_barrier()

    pltpu.sync_copy(deg_s.at[pl.ds(s * COLS, COLS)], d_v)
    cp_h.wait()

    def qbody(j, _):
        sl = pl.ds(j * 16, 16)
        y = _rsqrt3(d_v[sl] + 1.0)
        d_v[sl] = y
        h_v[sl] = y * h_v[sl]
        return 0

    lax.fori_loop(0, COLS // 16, qbody, 0)
    pltpu.sync_copy(h_v, q_s.at[pl.ds(s * COLS, COLS)])

    @pl.when(c == 0)
    def _():
        pltpu.sync_copy(d_v, dinv_out.at[s])

    plsc.subcore_barrier()

    pltpu.sync_copy(q_s, q_v)
    cp_sw.wait()
    cp_dw.wait()

    def gbody(j, _):
        for k in range(128 // 16):
            sl = pl.ds(k * 16, 16)
            qv_v[j, sl] = plsc.load_gather(q_v, [src_v[j, sl]])
        return 0

    lax.fori_loop(0, ER_W, gbody, 0)

    _fire_scatter_chunked(ER_W, lambda j: qv_v.at[j],
                          lambda j: dstw_v.at[j], acc_s, sem)
    plsc.subcore_barrier()

    pltpu.sync_copy(acc_s.at[pl.ds(s * COLS, COLS)], accp.at[c, s])


def _run_sc(dst2d, src2d, h2d):
    mesh = plsc.VectorSubcoreMesh(core_axis_name="c", subcore_axis_name="s",
                                  num_cores=NC, num_subcores=NS)
    return pl.kernel(
        _sc_body,
        out_type=(
            jax.ShapeDtypeStruct((NC, NS, COLS), jnp.float32),
            jax.ShapeDtypeStruct((NS, COLS), jnp.float32),
        ),
        mesh=mesh,
        compiler_params=pltpu.CompilerParams(needs_layout_passes=False),
        scratch_types=[
            pltpu.VMEM((COLS,), jnp.float32),
            pltpu.VMEM((128,), jnp.float32),
            pltpu.VMEM((ER_TILE, 128), jnp.int32),
            pltpu.VMEM((ER_W, 128), jnp.int32),
            pltpu.VMEM((ER_W, 128), jnp.int32),
            pltpu.VMEM((ER_W, 128), jnp.float32),
            pltpu.VMEM((NPAD,), jnp.float32),
            pltpu.VMEM((COLS,), jnp.float32),
            pltpu.VMEM((COLS,), jnp.float32),
            pltpu.VMEM_SHARED((NPAD,), jnp.float32),
            pltpu.VMEM_SHARED((NPAD,), jnp.float32),
            pltpu.VMEM_SHARED((NPAD,), jnp.float32),
            pltpu.SemaphoreType.DMA,
            pltpu.SemaphoreType.DMA,
        ],
    )(dst2d, src2d, h2d)


def _combine_body(x_ref, bf_ref, chn_ref, dinv_ref, acc0_ref, acc1_ref,
                  h_ref, bg_ref, o_ref):
    xb = x_ref[...]
    bf = bf_ref[...]
    iot = lax.broadcasted_iota(jnp.int32, (BN, G), 1).astype(jnp.float32)
    onehot = (bf == iot).astype(jnp.float32)
    chn_rows = jnp.dot(onehot, chn_ref[...], preferred_element_type=jnp.float32)
    dinv = dinv_ref[...]
    gcn = dinv * (acc0_ref[...] + acc1_ref[...] + dinv * h_ref[...]) \
        + bg_ref[0, 0]
    spa = jax.nn.sigmoid(gcn)
    o_ref[...] = xb * (1.0 + spa + chn_rows)


def _run_combine(x, bf, chn, dinv, acc0, acc1, h, bg):
    col = pl.BlockSpec((BN, 1), lambda i: (i, 0))
    return pl.pallas_call(
        _combine_body,
        grid=(GRID,),
        in_specs=[
            pl.BlockSpec((BN, C), lambda i: (i, 0)),
            col,
            pl.BlockSpec((G, C), lambda i: (0, 0)),
            col, col, col, col,
            pl.BlockSpec((1, 1), lambda i: (0, 0)),
        ],
        out_specs=pl.BlockSpec((BN, C), lambda i: (i, 0)),
        out_shape=jax.ShapeDtypeStruct((N, C), jnp.float32),
    )(x, bf, chn, dinv, acc0, acc1, h, bg)


@jax.jit
def kernel(x, batch, edge_index, W1, b1, W2, b2, Wg, bg):
    bf = batch.astype(jnp.float32).reshape(N, 1)
    h, chn = _run_stats(x, bf, W1, b1.reshape(1, G), W2, b2.reshape(1, C), Wg)

    src = jnp.pad(edge_index[0], (0, EPAD - E),
                  constant_values=PADNODE).reshape(ER_ALL, 128)
    dst = jnp.pad(edge_index[1], (0, EPAD - E),
                  constant_values=PADNODE).reshape(ER_ALL, 128)
    h2d = jnp.pad(h.reshape(N), (0, NPAD - N)).reshape(ROWS, COLS)

    accp, dinv = _run_sc(dst, src, h2d)

    accf = accp.reshape(NC, NPAD)
    acc0 = accf[0, :N].reshape(N, 1)
    acc1 = accf[1, :N].reshape(N, 1)
    dinv_c = dinv.reshape(NPAD)[:N].reshape(N, 1)

    return _run_combine(x, bf, chn, dinv_c, acc0, acc1, h,
                        bg.reshape(1, 1))


# speedup vs baseline: 72.9759x; 1.1458x over previous
import jax
import jax.numpy as jnp
from jax import lax
from jax.experimental import pallas as pl
from jax.experimental.pallas import tpu as pltpu
from jax.experimental.pallas import tpu_sc as plsc

NC, NS, LANES = 2, 16, 16
NW = NC * NS

N, C, E, G = 10000, 128, 320000, 64
BN = 2000
GRID = N // BN

COLS = 1024
NPAD = NS * COLS
PADNODE = NPAD - 1

ED = E // NS
EW = E // NW
CH_D = ED // 16
CH_W = EW // 16
ROWS_D = (ED + 127) // 128 + 1
ROWS_W = (EW + 127) // 128 + 1
FIRE = 16


def _stats_body(x_ref, b_ref, w1_ref, b1_ref, w2_ref, b2_ref, wg_ref,
                h_ref, chn_ref, sums_ref, cnt_ref):
    i = pl.program_id(0)
    xb = x_ref[...]
    bi = b_ref[...]
    onehot = (bi == lax.broadcasted_iota(jnp.int32, (BN, G), 1)
              ).astype(jnp.float32)
    psums = lax.dot_general(onehot, xb, (((0,), (0,)), ((), ())),
                            preferred_element_type=jnp.float32)
    pcnt = lax.dot_general(onehot, jnp.ones((BN, 1), jnp.float32),
                           (((0,), (0,)), ((), ())),
                           preferred_element_type=jnp.float32)

    @pl.when(i == 0)
    def _():
        sums_ref[...] = jnp.zeros_like(sums_ref)
        cnt_ref[...] = jnp.zeros_like(cnt_ref)

    sums_ref[...] += psums
    cnt_ref[...] += pcnt
    h_ref[...] = jnp.dot(xb, wg_ref[...], preferred_element_type=jnp.float32)

    @pl.when(i == GRID - 1)
    def _():
        means = sums_ref[...] / jnp.maximum(cnt_ref[...], 1.0)
        t = jnp.maximum(
            jnp.dot(means, w1_ref[...], preferred_element_type=jnp.float32)
            + b1_ref[...], 0.0)
        chn_ref[...] = jax.nn.sigmoid(
            jnp.dot(t, w2_ref[...], preferred_element_type=jnp.float32)
            + b2_ref[...])


def _run_stats(x, bi, W1, b1, W2, b2, Wg):
    return pl.pallas_call(
        _stats_body,
        grid=(GRID,),
        in_specs=[
            pl.BlockSpec((BN, C), lambda i: (i, 0)),
            pl.BlockSpec((BN, 1), lambda i: (i, 0)),
            pl.BlockSpec((C, G), lambda i: (0, 0)),
            pl.BlockSpec((1, G), lambda i: (0, 0)),
            pl.BlockSpec((G, C), lambda i: (0, 0)),
            pl.BlockSpec((1, C), lambda i: (0, 0)),
            pl.BlockSpec((C, 1), lambda i: (0, 0)),
        ],
        out_specs=[
            pl.BlockSpec((BN, 1), lambda i: (i, 0)),
            pl.BlockSpec((G, C), lambda i: (0, 0)),
        ],
        out_shape=[
            jax.ShapeDtypeStruct((N, 1), jnp.float32),
            jax.ShapeDtypeStruct((G, C), jnp.float32),
        ],
        scratch_shapes=[
            pltpu.VMEM((G, C), jnp.float32),
            pltpu.VMEM((G, 1), jnp.float32),
        ],
    )(x, bi, W1, b1, W2, b2, Wg)


def _rsqrt3(d):
    yi = plsc.bitcast(d, jnp.int32)
    yi = 0x5F3759DF - lax.shift_right_arithmetic(yi, 1)
    y = plsc.bitcast(yi, jnp.float32)
    half = 0.5 * d
    y = y * (1.5 - half * y * y)
    y = y * (1.5 - half * y * y)
    y = y * (1.5 - half * y * y)
    return y


def _fire_scatter(n_rows, vals_row, idx2d, target, sem):
    for base in range(0, n_rows, FIRE):
        k = min(FIRE, n_rows - base)
        descs = [
            pltpu.async_copy(vals_row(base + j),
                             target.at[idx2d.at[base + j]], sem, add=True)
            for j in range(k)
        ]
        for dsc in descs:
            dsc.wait()


def _pad_tail(buf1d, start, n_chunks):
    padv = jnp.full((16,), PADNODE, jnp.int32)
    for t in range(n_chunks):
        buf1d[pl.ds(start + t * 16, 16)] = padv


def _sc_body(er, h1, accp, dinv_out,
             zeros_v, ones_v, dstd1_v, dstd2_v, src1_v, dstw1_v, dstw2_v,
             qv_v, q_v, d_v, h_v, deg_s, q_s, acc_s, sem, sem2):
    c = lax.axis_index("c")
    s = lax.axis_index("s")
    wid = c * NS + s

    cp_dd = pltpu.async_copy(er.at[pl.ds(E + s * ED, ED)],
                             dstd1_v.at[pl.ds(0, ED)], sem2)
    cp_sw = pltpu.async_copy(er.at[pl.ds(wid * EW, EW)],
                             src1_v.at[pl.ds(0, EW)], sem2)
    cp_dw = pltpu.async_copy(er.at[pl.ds(E + wid * EW, EW)],
                             dstw1_v.at[pl.ds(0, EW)], sem2)

    def zbody(j, _):
        zeros_v[pl.ds(j * 16, 16)] = jnp.zeros((16,), jnp.float32)
        return 0

    lax.fori_loop(0, COLS // 16, zbody, 0)
    for k in range(128 // 16):
        ones_v[pl.ds(k * 16, 16)] = jnp.ones((16,), jnp.float32)

    pltpu.sync_copy(zeros_v, deg_s.at[pl.ds(s * COLS, COLS)])
    pltpu.sync_copy(zeros_v, acc_s.at[pl.ds(s * COLS, COLS)])
    plsc.subcore_barrier()

    cp_dd.wait()
    _pad_tail(dstd1_v, ED, ROWS_D * 8 - CH_D)

    def cbody(j, _):
        dstd2_v[j >> 3, pl.ds((j & 7) * 16, 16)] = dstd1_v[pl.ds(j * 16, 16)]
        return 0

    lax.fori_loop(0, ROWS_D * 8, cbody, 0)
    _fire_scatter(ROWS_D, lambda j: ones_v, dstd2_v, deg_s, sem)
    plsc.subcore_barrier()

    pltpu.sync_copy(deg_s.at[pl.ds(s * COLS, COLS)], d_v)

    @pl.when(s < (N // COLS))
    def _():
        pltpu.sync_copy(h1.at[pl.ds(s * COLS, COLS)], h_v)

    @pl.when(s == (N // COLS))
    def _():
        rem = N - (N // COLS) * COLS
        pltpu.sync_copy(h1.at[pl.ds((N // COLS) * COLS, rem)],
                        h_v.at[pl.ds(0, rem)])

    def qbody(j, _):
        sl = pl.ds(j * 16, 16)
        y = _rsqrt3(d_v[sl] + 1.0)
        d_v[sl] = y
        h_v[sl] = y * h_v[sl]
        return 0

    lax.fori_loop(0, COLS // 16, qbody, 0)
    pltpu.sync_copy(h_v, q_s.at[pl.ds(s * COLS, COLS)])

    @pl.when(c == 0)
    def _():
        pltpu.sync_copy(d_v, dinv_out.at[pl.ds(s * COLS, COLS)])

    plsc.subcore_barrier()

    pltpu.sync_copy(q_s, q_v)
    cp_sw.wait()
    cp_dw.wait()
    _pad_tail(src1_v, EW, ROWS_W * 8 - CH_W)
    _pad_tail(dstw1_v, EW, ROWS_W * 8 - CH_W)

    def gbody(j, _):
        sl16 = pl.ds((j & 7) * 16, 16)
        row = j >> 3
        qv_v[row, sl16] = plsc.load_gather(q_v, [src1_v[pl.ds(j * 16, 16)]])
        dstw2_v[row, sl16] = dstw1_v[pl.ds(j * 16, 16)]
        return 0

    lax.fori_loop(0, ROWS_W * 8, gbody, 0)
    _fire_scatter(ROWS_W, lambda j: qv_v.at[j], dstw2_v, acc_s, sem)
    plsc.subcore_barrier()

    pltpu.sync_copy(acc_s.at[pl.ds(s * COLS, COLS)],
                    accp.at[c, pl.ds(s * COLS, COLS)])


def _run_sc(er, h1):
    mesh = plsc.VectorSubcoreMesh(core_axis_name="c", subcore_axis_name="s",
                                  num_cores=NC, num_subcores=NS)
    return pl.kernel(
        _sc_body,
        out_type=(
            jax.ShapeDtypeStruct((NC, NPAD), jnp.float32),
            jax.ShapeDtypeStruct((NPAD,), jnp.float32),
        ),
        mesh=mesh,
        compiler_params=pltpu.CompilerParams(needs_layout_passes=False),
        scratch_types=[
            pltpu.VMEM((COLS,), jnp.float32),
            pltpu.VMEM((128,), jnp.float32),
            pltpu.VMEM((ROWS_D * 128,), jnp.int32),
            pltpu.VMEM((ROWS_D, 128), jnp.int32),
            pltpu.VMEM((ROWS_W * 128,), jnp.int32),
            pltpu.VMEM((ROWS_W * 128,), jnp.int32),
            pltpu.VMEM((ROWS_W, 128), jnp.int32),
            pltpu.VMEM((ROWS_W, 128), jnp.float32),
            pltpu.VMEM((NPAD,), jnp.float32),
            pltpu.VMEM((COLS,), jnp.float32),
            pltpu.VMEM((COLS,), jnp.float32),
            pltpu.VMEM_SHARED((NPAD,), jnp.float32),
            pltpu.VMEM_SHARED((NPAD,), jnp.float32),
            pltpu.VMEM_SHARED((NPAD,), jnp.float32),
            pltpu.SemaphoreType.DMA,
            pltpu.SemaphoreType.DMA,
        ],
    )(er, h1)


def _combine_body(x_ref, b_ref, chn_ref, dinv_ref, acc0_ref, acc1_ref,
                  h_ref, bg_ref, o_ref):
    xb = x_ref[...]
    bi = b_ref[...]
    onehot = (bi == lax.broadcasted_iota(jnp.int32, (BN, G), 1)
              ).astype(jnp.float32)
    chn_rows = jnp.dot(onehot, chn_ref[...], preferred_element_type=jnp.float32)
    dinv = dinv_ref[...]
    gcn = dinv * (acc0_ref[...] + acc1_ref[...] + dinv * h_ref[...]) \
        + bg_ref[0, 0]
    spa = jax.nn.sigmoid(gcn)
    o_ref[...] = xb * (1.0 + spa + chn_rows)


def _run_combine(x, bi, chn, dinv, acc0, acc1, h, bg):
    col = pl.BlockSpec((BN, 1), lambda i: (i, 0))
    return pl.pallas_call(
        _combine_body,
        grid=(GRID,),
        in_specs=[
            pl.BlockSpec((BN, C), lambda i: (i, 0)),
            col,
            pl.BlockSpec((G, C), lambda i: (0, 0)),
            col, col, col, col,
            pl.BlockSpec((1, 1), lambda i: (0, 0)),
        ],
        out_specs=pl.BlockSpec((BN, C), lambda i: (i, 0)),
        out_shape=jax.ShapeDtypeStruct((N, C), jnp.float32),
    )(x, bi, chn, dinv, acc0, acc1, h, bg)


@jax.jit
def kernel(x, batch, edge_index, W1, b1, W2, b2, Wg, bg):
    bi = batch.reshape(N, 1)
    h, chn = _run_stats(x, bi, W1, b1.reshape(1, G), W2, b2.reshape(1, C), Wg)

    accp, dinv = _run_sc(edge_index.reshape(2 * E), h.reshape(N))

    acc0 = accp[0, :N].reshape(N, 1)
    acc1 = accp[1, :N].reshape(N, 1)
    dinv_c = dinv[:N].reshape(N, 1)

    return _run_combine(x, bi, chn, dinv_c, acc0, acc1, h, bg.reshape(1, 1))


# speedup vs baseline: 97.7688x; 1.3397x over previous
import jax
import jax.numpy as jnp
from jax import lax
from jax.experimental import pallas as pl
from jax.experimental.pallas import tpu as pltpu
from jax.experimental.pallas import tpu_sc as plsc

NC, NS, LANES = 2, 16, 16
NW = NC * NS

N, C, E, G = 10000, 128, 320000, 64
BN = 2048
GRID = (N + BN - 1) // BN

COLS = 1024
NPAD = NS * COLS
PADNODE = NPAD - 1
HROWS = N // COLS
HREM = N - HROWS * COLS

ED = E // NS
EW = E // NW
CH_D = ED // 16
CH_W = EW // 16
ROWS_D = CH_D // 8 + 1
ROWS_W = CH_W // 8 + 1
BUF_D = (ED // 128 + 2) * 128
BUF_W = (EW // 128 + 2) * 128
FIRE = 16


def _stats_body(x_ref, b_ref, w1_ref, b1_ref, w2_ref, b2_ref, wg_ref,
                h_ref, chn_ref, sums_ref, cnt_ref):
    i = pl.program_id(0)
    xb = x_ref[...]
    bi = b_ref[...]
    rid = lax.broadcasted_iota(jnp.int32, (BN, 1), 0) + i * BN
    valid = rid < N
    onehot = jnp.where(
        valid & (bi == lax.broadcasted_iota(jnp.int32, (BN, G), 1)),
        1.0, 0.0)
    xbm = jnp.where(valid, xb, 0.0)
    psums = lax.dot_general(onehot, xbm, (((0,), (0,)), ((), ())),
                            preferred_element_type=jnp.float32)
    pcnt = lax.dot_general(onehot, jnp.ones((BN, 1), jnp.float32),
                           (((0,), (0,)), ((), ())),
                           preferred_element_type=jnp.float32)

    @pl.when(i == 0)
    def _():
        sums_ref[...] = jnp.zeros_like(sums_ref)
        cnt_ref[...] = jnp.zeros_like(cnt_ref)

    sums_ref[...] += psums
    cnt_ref[...] += pcnt
    hv = jnp.dot(xb, wg_ref[...], preferred_element_type=jnp.float32)
    h_ref[pl.ds(i * BN, BN)] = hv.reshape(BN)

    @pl.when(i == GRID - 1)
    def _():
        means = sums_ref[...] / jnp.maximum(cnt_ref[...], 1.0)
        t = jnp.maximum(
            jnp.dot(means, w1_ref[...], preferred_element_type=jnp.float32)
            + b1_ref[...], 0.0)
        chn_ref[...] = jax.nn.sigmoid(
            jnp.dot(t, w2_ref[...], preferred_element_type=jnp.float32)
            + b2_ref[...])


def _run_stats(x, bi, W1, b1, W2, b2, Wg):
    return pl.pallas_call(
        _stats_body,
        grid=(GRID,),
        in_specs=[
            pl.BlockSpec((BN, C), lambda i: (i, 0)),
            pl.BlockSpec((BN, 1), lambda i: (i, 0)),
            pl.BlockSpec((C, G), lambda i: (0, 0)),
            pl.BlockSpec((1, G), lambda i: (0, 0)),
            pl.BlockSpec((G, C), lambda i: (0, 0)),
            pl.BlockSpec((1, C), lambda i: (0, 0)),
            pl.BlockSpec((C, 1), lambda i: (0, 0)),
        ],
        out_specs=[
            pl.BlockSpec((NPAD,), lambda i: (0,)),
            pl.BlockSpec((G, C), lambda i: (0, 0)),
        ],
        out_shape=[
            jax.ShapeDtypeStruct((NPAD,), jnp.float32),
            jax.ShapeDtypeStruct((G, C), jnp.float32),
        ],
        scratch_shapes=[
            pltpu.VMEM((G, C), jnp.float32),
            pltpu.VMEM((G, 1), jnp.float32),
        ],
    )(x, bi, W1, b1, W2, b2, Wg)


def _rsqrt3(d):
    yi = plsc.bitcast(d, jnp.int32)
    yi = 0x5F3759DF - lax.shift_right_arithmetic(yi, 1)
    y = plsc.bitcast(yi, jnp.float32)
    half = 0.5 * d
    y = y * (1.5 - half * y * y)
    y = y * (1.5 - half * y * y)
    y = y * (1.5 - half * y * y)
    return y


def _fire_scatter(n_rows, vals_row, idx2d, target, sem):
    for base in range(0, n_rows, FIRE):
        k = min(FIRE, n_rows - base)
        descs = [
            pltpu.async_copy(vals_row(base + j),
                             target.at[idx2d.at[base + j]], sem, add=True)
            for j in range(k)
        ]
        for dsc in descs:
            dsc.wait()


def _sc_body(ei, h2d, acc0_out, acc1_out, dinv_out,
             zeros_v, ones_v, degb_v, dstd2_v, accb_v, dstw2_v,
             qv_v, q_v, d_v, h_v, deg_s, q_s, acc_s,
             sem, semd, semw, semh):
    c = lax.axis_index("c")
    s = lax.axis_index("s")
    wid = c * NS + s
    padv = jnp.full((16,), PADNODE, jnp.int32)

    pre_d = (s * ED) % 128
    ab_d = pl.multiple_of(s * ED - pre_d, 128)
    pre_w = (wid * EW) % 128
    ab_w = pl.multiple_of(wid * EW - pre_w, 128)

    @pl.when(s < NS - 1)
    def _():
        pltpu.async_copy(ei.at[:, pl.ds(ab_d, BUF_D)], degb_v, semd)

    @pl.when(s == NS - 1)
    def _():
        pltpu.async_copy(ei.at[:, pl.ds(ab_d, BUF_D - 128)],
                         degb_v.at[:, pl.ds(0, BUF_D - 128)], semd)

    @pl.when(wid < NW - 1)
    def _():
        pltpu.async_copy(ei.at[:, pl.ds(ab_w, BUF_W)], accb_v, semw)

    @pl.when(wid == NW - 1)
    def _():
        pltpu.async_copy(ei.at[:, pl.ds(ab_w, BUF_W - 128)],
                         accb_v.at[:, pl.ds(0, BUF_W - 128)], semw)

    def zbody(j, _):
        zeros_v[pl.ds(j * 16, 16)] = jnp.zeros((16,), jnp.float32)
        return 0

    lax.fori_loop(0, COLS // 16, zbody, 0)
    for k in range(128 // 16):
        ones_v[pl.ds(k * 16, 16)] = jnp.ones((16,), jnp.float32)

    pltpu.sync_copy(zeros_v, deg_s.at[pl.ds(s * COLS, COLS)])
    pltpu.sync_copy(zeros_v, acc_s.at[pl.ds(s * COLS, COLS)])

    pltpu.async_copy(h2d.at[pl.ds(s * COLS, COLS)], h_v, semh)

    plsc.subcore_barrier()

    @pl.when(s < NS - 1)
    def _():
        pltpu.make_async_copy(ei.at[:, pl.ds(ab_d, BUF_D)], degb_v,
                              semd).wait()

    @pl.when(s == NS - 1)
    def _():
        pltpu.make_async_copy(ei.at[:, pl.ds(ab_d, BUF_D - 128)],
                              degb_v.at[:, pl.ds(0, BUF_D - 128)],
                              semd).wait()

    def cbody(j, _):
        off = jnp.minimum(pre_d + j * 16, BUF_D - 16)
        v = jnp.where(j < CH_D, degb_v[1, pl.ds(off, 16)], padv)
        dstd2_v[j >> 3, pl.ds((j & 7) * 16, 16)] = v
        return 0

    lax.fori_loop(0, ROWS_D * 8, cbody, 0)
    _fire_scatter(ROWS_D, lambda j: ones_v, dstd2_v, deg_s, sem)
    plsc.subcore_barrier()

    pltpu.sync_copy(deg_s.at[pl.ds(s * COLS, COLS)], d_v)

    pltpu.make_async_copy(h2d.at[pl.ds(0, COLS)], h_v, semh).wait()

    def qbody(j, _):
        sl = pl.ds(j * 16, 16)
        y = _rsqrt3(d_v[sl] + 1.0)
        d_v[sl] = y
        h_v[sl] = y * h_v[sl]
        return 0

    lax.fori_loop(0, COLS // 16, qbody, 0)
    pltpu.sync_copy(h_v, q_s.at[pl.ds(s * COLS, COLS)])

    @pl.when(c == 0)
    def _():
        pltpu.sync_copy(d_v, dinv_out.at[pl.ds(s * COLS, COLS)])

    plsc.subcore_barrier()

    pltpu.sync_copy(q_s, q_v)
    @pl.when(wid < NW - 1)
    def _():
        pltpu.make_async_copy(ei.at[:, pl.ds(ab_w, BUF_W)], accb_v,
                              semw).wait()

    @pl.when(wid == NW - 1)
    def _():
        pltpu.make_async_copy(ei.at[:, pl.ds(ab_w, BUF_W - 128)],
                              accb_v.at[:, pl.ds(0, BUF_W - 128)],
                              semw).wait()

    def gbody(j, _):
        off = jnp.minimum(pre_w + j * 16, BUF_W - 16)
        sl16 = pl.ds((j & 7) * 16, 16)
        row = j >> 3
        sidx = jnp.where(j < CH_W, accb_v[0, pl.ds(off, 16)], padv)
        didx = jnp.where(j < CH_W, accb_v[1, pl.ds(off, 16)], padv)
        qv_v[row, sl16] = plsc.load_gather(q_v, [sidx])
        dstw2_v[row, sl16] = didx
        return 0

    lax.fori_loop(0, ROWS_W * 8, gbody, 0)
    _fire_scatter(ROWS_W, lambda j: qv_v.at[j], dstw2_v, acc_s, sem)
    plsc.subcore_barrier()

    @pl.when(c == 0)
    def _():
        pltpu.sync_copy(acc_s.at[pl.ds(s * COLS, COLS)],
                        acc0_out.at[pl.ds(s * COLS, COLS)])

    @pl.when(c == 1)
    def _():
        pltpu.sync_copy(acc_s.at[pl.ds(s * COLS, COLS)],
                        acc1_out.at[pl.ds(s * COLS, COLS)])


def _run_sc(ei, h2d):
    mesh = plsc.VectorSubcoreMesh(core_axis_name="c", subcore_axis_name="s",
                                  num_cores=NC, num_subcores=NS)
    return pl.kernel(
        _sc_body,
        out_type=(
            jax.ShapeDtypeStruct((NPAD,), jnp.float32),
            jax.ShapeDtypeStruct((NPAD,), jnp.float32),
            jax.ShapeDtypeStruct((NPAD,), jnp.float32),
        ),
        mesh=mesh,
        compiler_params=pltpu.CompilerParams(needs_layout_passes=False),
        scratch_types=[
            pltpu.VMEM((COLS,), jnp.float32),
            pltpu.VMEM((128,), jnp.float32),
            pltpu.VMEM((2, BUF_D), jnp.int32),
            pltpu.VMEM((ROWS_D, 128), jnp.int32),
            pltpu.VMEM((2, BUF_W), jnp.int32),
            pltpu.VMEM((ROWS_W, 128), jnp.int32),
            pltpu.VMEM((ROWS_W, 128), jnp.float32),
            pltpu.VMEM((NPAD,), jnp.float32),
            pltpu.VMEM((COLS,), jnp.float32),
            pltpu.VMEM((COLS,), jnp.float32),
            pltpu.VMEM_SHARED((NPAD,), jnp.float32),
            pltpu.VMEM_SHARED((NPAD,), jnp.float32),
            pltpu.VMEM_SHARED((NPAD,), jnp.float32),
            pltpu.SemaphoreType.DMA,
            pltpu.SemaphoreType.DMA,
            pltpu.SemaphoreType.DMA,
            pltpu.SemaphoreType.DMA,
        ],
    )(ei, h2d)


def _combine_body(x_ref, b_ref, chn_ref, dinv_ref, acc0_ref, acc1_ref,
                  h1_ref, bg_ref, o_ref):
    i = pl.program_id(0)
    xb = x_ref[...]
    bi = b_ref[...]
    onehot = (bi == lax.broadcasted_iota(jnp.int32, (BN, G), 1)
              ).astype(jnp.float32)
    chn_rows = jnp.dot(onehot, chn_ref[...], preferred_element_type=jnp.float32)
    sl = pl.ds(i * BN, BN)
    dinv = dinv_ref[sl]
    gcn = dinv * (acc0_ref[sl] + acc1_ref[sl]) + dinv * dinv * h1_ref[sl]
    spa = jax.nn.sigmoid(gcn + bg_ref[0, 0]).reshape(BN, 1)
    o_ref[...] = xb * (1.0 + spa + chn_rows)


def _run_combine(x, bi, chn, dinv, acc0, acc1, h, bg):
    full = pl.BlockSpec((NPAD,), lambda i: (0,))
    return pl.pallas_call(
        _combine_body,
        grid=(GRID,),
        in_specs=[
            pl.BlockSpec((BN, C), lambda i: (i, 0)),
            pl.BlockSpec((BN, 1), lambda i: (i, 0)),
            pl.BlockSpec((G, C), lambda i: (0, 0)),
            full, full, full, full,
            pl.BlockSpec((1, 1), lambda i: (0, 0)),
        ],
        out_specs=pl.BlockSpec((BN, C), lambda i: (i, 0)),
        out_shape=jax.ShapeDtypeStruct((N, C), jnp.float32),
    )(x, bi, chn, dinv, acc0, acc1, h, bg)


@jax.jit
def kernel(x, batch, edge_index, W1, b1, W2, b2, Wg, bg):
    bi = batch.reshape(N, 1)
    h, chn = _run_stats(x, bi, W1, b1.reshape(1, G), W2, b2.reshape(1, C), Wg)

    acc0, acc1, dinv = _run_sc(edge_index, h)

    return _run_combine(x, bi, chn, dinv, acc0, acc1, h, bg.reshape(1, 1))


# speedup vs baseline: 108.6939x; 1.1117x over previous
import jax
import jax.numpy as jnp
from jax import lax
from jax.experimental import pallas as pl
from jax.experimental.pallas import tpu as pltpu
from jax.experimental.pallas import tpu_sc as plsc

NC, NS, LANES = 2, 16, 16
NW = NC * NS

N, C, E, G = 10000, 128, 320000, 64
BN = 2048
GRID = (N + BN - 1) // BN

COLS = 1024
NPAD = NS * COLS
PADNODE = NPAD - 1
HROWS = N // COLS
HREM = N - HROWS * COLS

ED = E // NS
EW = E // NW
CH_D = ED // 16
CH_W = EW // 16
ROWS_D = CH_D // 8 + 1
ROWS_W = CH_W // 8 + 1
BUF_D = (ED // 128 + 2) * 128
BUF_W = (EW // 128 + 2) * 128
FIRE = 16


def _stats_body(x_ref, b_ref, w1_ref, b1_ref, w2_ref, b2_ref, wg_ref,
                h_ref, chn_ref, sums_ref, cnt_ref):
    i = pl.program_id(0)
    xb = x_ref[...]
    bi = b_ref[...].reshape(BN, 1)
    hit = bi == lax.broadcasted_iota(jnp.int32, (BN, G), 1)

    @pl.when(i == 0)
    def _():
        sums_ref[...] = jnp.zeros_like(sums_ref)
        cnt_ref[...] = jnp.zeros_like(cnt_ref)

    def accum(onehot, xv):
        psums = lax.dot_general(onehot, xv, (((0,), (0,)), ((), ())),
                                preferred_element_type=jnp.float32)
        pcnt = lax.dot_general(onehot, jnp.ones((BN, 1), jnp.float32),
                               (((0,), (0,)), ((), ())),
                               preferred_element_type=jnp.float32)
        sums_ref[...] += psums
        cnt_ref[...] += pcnt

    @pl.when(i < GRID - 1)
    def _():
        accum(jnp.where(hit, 1.0, 0.0), xb)

    @pl.when(i == GRID - 1)
    def _():
        rid = lax.broadcasted_iota(jnp.int32, (BN, 1), 0) + i * BN
        valid = rid < N
        accum(jnp.where(valid & hit, 1.0, 0.0), jnp.where(valid, xb, 0.0))

    hv = jnp.dot(xb, wg_ref[...], preferred_element_type=jnp.float32)
    h_ref[pl.ds(i * BN, BN)] = hv.reshape(BN)

    @pl.when(i == GRID - 1)
    def _():
        means = sums_ref[...] / jnp.maximum(cnt_ref[...], 1.0)
        t = jnp.maximum(
            jnp.dot(means, w1_ref[...], preferred_element_type=jnp.float32)
            + b1_ref[...].reshape(1, G), 0.0)
        chn_ref[...] = jax.nn.sigmoid(
            jnp.dot(t, w2_ref[...], preferred_element_type=jnp.float32)
            + b2_ref[...].reshape(1, C))


def _run_stats(x, bi, W1, b1, W2, b2, Wg):
    return pl.pallas_call(
        _stats_body,
        grid=(GRID,),
        in_specs=[
            pl.BlockSpec((BN, C), lambda i: (i, 0)),
            pl.BlockSpec((BN,), lambda i: (i,)),
            pl.BlockSpec((C, G), lambda i: (0, 0)),
            pl.BlockSpec((G,), lambda i: (0,)),
            pl.BlockSpec((G, C), lambda i: (0, 0)),
            pl.BlockSpec((C,), lambda i: (0,)),
            pl.BlockSpec((C, 1), lambda i: (0, 0)),
        ],
        out_specs=[
            pl.BlockSpec((NPAD,), lambda i: (0,)),
            pl.BlockSpec((G, C), lambda i: (0, 0)),
        ],
        out_shape=[
            jax.ShapeDtypeStruct((NPAD,), jnp.float32),
            jax.ShapeDtypeStruct((G, C), jnp.float32),
        ],
        scratch_shapes=[
            pltpu.VMEM((G, C), jnp.float32),
            pltpu.VMEM((G, 1), jnp.float32),
        ],
    )(x, bi, W1, b1, W2, b2, Wg)


def _rsqrt3(d):
    yi = plsc.bitcast(d, jnp.int32)
    yi = 0x5F3759DF - lax.shift_right_arithmetic(yi, 1)
    y = plsc.bitcast(yi, jnp.float32)
    half = 0.5 * d
    y = y * (1.5 - half * y * y)
    y = y * (1.5 - half * y * y)
    y = y * (1.5 - half * y * y)
    return y


def _fire_scatter(n_rows, vals_row, idx2d, target, sem):
    for base in range(0, n_rows, FIRE):
        k = min(FIRE, n_rows - base)
        descs = [
            pltpu.async_copy(vals_row(base + j),
                             target.at[idx2d.at[base + j]], sem, add=True)
            for j in range(k)
        ]
        for dsc in descs:
            dsc.wait()


def _sc_body(ei, h2d, acc0_out, acc1_out, dinv_out,
             zeros_v, ones_v, degb_v, dstd2_v, accb_v, dstw2_v,
             qv_v, q_v, d_v, h_v, deg_s, q_s, acc_s,
             sem, semd, semw, semh):
    c = lax.axis_index("c")
    s = lax.axis_index("s")
    wid = c * NS + s
    padv = jnp.full((16,), PADNODE, jnp.int32)

    pre_d = (s * ED) % 128
    ab_d = pl.multiple_of(s * ED - pre_d, 128)
    pre_w = (wid * EW) % 128
    ab_w = pl.multiple_of(wid * EW - pre_w, 128)

    @pl.when(s < NS - 1)
    def _():
        pltpu.async_copy(ei.at[:, pl.ds(ab_d, BUF_D)], degb_v, semd)

    @pl.when(s == NS - 1)
    def _():
        pltpu.async_copy(ei.at[:, pl.ds(ab_d, BUF_D - 128)],
                         degb_v.at[:, pl.ds(0, BUF_D - 128)], semd)

    @pl.when(wid < NW - 1)
    def _():
        pltpu.async_copy(ei.at[:, pl.ds(ab_w, BUF_W)], accb_v, semw)

    @pl.when(wid == NW - 1)
    def _():
        pltpu.async_copy(ei.at[:, pl.ds(ab_w, BUF_W - 128)],
                         accb_v.at[:, pl.ds(0, BUF_W - 128)], semw)

    def zbody(j, _):
        zeros_v[pl.ds(j * 16, 16)] = jnp.zeros((16,), jnp.float32)
        return 0

    lax.fori_loop(0, COLS // 16, zbody, 0)
    for k in range(128 // 16):
        ones_v[pl.ds(k * 16, 16)] = jnp.ones((16,), jnp.float32)

    pltpu.sync_copy(zeros_v, deg_s.at[pl.ds(s * COLS, COLS)])
    pltpu.sync_copy(zeros_v, acc_s.at[pl.ds(s * COLS, COLS)])

    pltpu.async_copy(h2d.at[pl.ds(s * COLS, COLS)], h_v, semh)

    plsc.subcore_barrier()

    @pl.when(s < NS - 1)
    def _():
        pltpu.make_async_copy(ei.at[:, pl.ds(ab_d, BUF_D)], degb_v,
                              semd).wait()

    @pl.when(s == NS - 1)
    def _():
        pltpu.make_async_copy(ei.at[:, pl.ds(ab_d, BUF_D - 128)],
                              degb_v.at[:, pl.ds(0, BUF_D - 128)],
                              semd).wait()

    def cbody(j, _):
        off = jnp.minimum(pre_d + j * 16, BUF_D - 16)
        v = jnp.where(j < CH_D, degb_v[1, pl.ds(off, 16)], padv)
        dstd2_v[j >> 3, pl.ds((j & 7) * 16, 16)] = v
        return 0

    lax.fori_loop(0, ROWS_D * 8, cbody, 0)
    _fire_scatter(ROWS_D, lambda j: ones_v, dstd2_v, deg_s, sem)
    plsc.subcore_barrier()

    pltpu.sync_copy(deg_s.at[pl.ds(s * COLS, COLS)], d_v)

    pltpu.make_async_copy(h2d.at[pl.ds(0, COLS)], h_v, semh).wait()

    def qbody(j, _):
        sl = pl.ds(j * 16, 16)
        y = _rsqrt3(d_v[sl] + 1.0)
        d_v[sl] = y
        h_v[sl] = y * h_v[sl]
        return 0

    lax.fori_loop(0, COLS // 16, qbody, 0)
    pltpu.sync_copy(h_v, q_s.at[pl.ds(s * COLS, COLS)])

    @pl.when(c == 0)
    def _():
        pltpu.sync_copy(d_v, dinv_out.at[pl.ds(s * COLS, COLS)])

    plsc.subcore_barrier()

    pltpu.sync_copy(q_s, q_v)
    @pl.when(wid < NW - 1)
    def _():
        pltpu.make_async_copy(ei.at[:, pl.ds(ab_w, BUF_W)], accb_v,
                              semw).wait()

    @pl.when(wid == NW - 1)
    def _():
        pltpu.make_async_copy(ei.at[:, pl.ds(ab_w, BUF_W - 128)],
                              accb_v.at[:, pl.ds(0, BUF_W - 128)],
                              semw).wait()

    def gbody(j, _):
        off = jnp.minimum(pre_w + j * 16, BUF_W - 16)
        sl16 = pl.ds((j & 7) * 16, 16)
        row = j >> 3
        sidx = jnp.where(j < CH_W, accb_v[0, pl.ds(off, 16)], padv)
        didx = jnp.where(j < CH_W, accb_v[1, pl.ds(off, 16)], padv)
        qv_v[row, sl16] = plsc.load_gather(q_v, [sidx])
        dstw2_v[row, sl16] = didx
        return 0

    lax.fori_loop(0, ROWS_W * 8, gbody, 0)
    _fire_scatter(ROWS_W, lambda j: qv_v.at[j], dstw2_v, acc_s, sem)
    plsc.subcore_barrier()

    @pl.when(c == 0)
    def _():
        pltpu.sync_copy(acc_s.at[pl.ds(s * COLS, COLS)],
                        acc0_out.at[pl.ds(s * COLS, COLS)])

    @pl.when(c == 1)
    def _():
        pltpu.sync_copy(acc_s.at[pl.ds(s * COLS, COLS)],
                        acc1_out.at[pl.ds(s * COLS, COLS)])


def _run_sc(ei, h2d):
    mesh = plsc.VectorSubcoreMesh(core_axis_name="c", subcore_axis_name="s",
                                  num_cores=NC, num_subcores=NS)
    return pl.kernel(
        _sc_body,
        out_type=(
            jax.ShapeDtypeStruct((NPAD,), jnp.float32),
            jax.ShapeDtypeStruct((NPAD,), jnp.float32),
            jax.ShapeDtypeStruct((NPAD,), jnp.float32),
        ),
        mesh=mesh,
        compiler_params=pltpu.CompilerParams(needs_layout_passes=False),
        scratch_types=[
            pltpu.VMEM((COLS,), jnp.float32),
            pltpu.VMEM((128,), jnp.float32),
            pltpu.VMEM((2, BUF_D), jnp.int32),
            pltpu.VMEM((ROWS_D, 128), jnp.int32),
            pltpu.VMEM((2, BUF_W), jnp.int32),
            pltpu.VMEM((ROWS_W, 128), jnp.int32),
            pltpu.VMEM((ROWS_W, 128), jnp.float32),
            pltpu.VMEM((NPAD,), jnp.float32),
            pltpu.VMEM((COLS,), jnp.float32),
            pltpu.VMEM((COLS,), jnp.float32),
            pltpu.VMEM_SHARED((NPAD,), jnp.float32),
            pltpu.VMEM_SHARED((NPAD,), jnp.float32),
            pltpu.VMEM_SHARED((NPAD,), jnp.float32),
            pltpu.SemaphoreType.DMA,
            pltpu.SemaphoreType.DMA,
            pltpu.SemaphoreType.DMA,
            pltpu.SemaphoreType.DMA,
        ],
    )(ei, h2d)


def _combine_body(x_ref, b_ref, chn_ref, dinv_ref, acc0_ref, acc1_ref,
                  h1_ref, bg_ref, o_ref):
    i = pl.program_id(0)
    xb = x_ref[...]
    bi = b_ref[...].reshape(BN, 1)
    onehot = (bi == lax.broadcasted_iota(jnp.int32, (BN, G), 1)
              ).astype(jnp.float32)
    chn_rows = jnp.dot(onehot, chn_ref[...], preferred_element_type=jnp.float32)
    sl = pl.ds(i * BN, BN)
    dinv = dinv_ref[sl]
    gcn = dinv * (acc0_ref[sl] + acc1_ref[sl]) + dinv * dinv * h1_ref[sl]
    spa = jax.nn.sigmoid(gcn + bg_ref[...]).reshape(BN, 1)
    o_ref[...] = xb * (1.0 + spa + chn_rows)


def _run_combine(x, bi, chn, dinv, acc0, acc1, h, bg):
    full = pl.BlockSpec((NPAD,), lambda i: (0,))
    return pl.pallas_call(
        _combine_body,
        grid=(GRID,),
        in_specs=[
            pl.BlockSpec((BN, C), lambda i: (i, 0)),
            pl.BlockSpec((BN,), lambda i: (i,)),
            pl.BlockSpec((G, C), lambda i: (0, 0)),
            full, full, full, full,
            pl.BlockSpec((1,), lambda i: (0,)),
        ],
        out_specs=pl.BlockSpec((BN, C), lambda i: (i, 0)),
        out_shape=jax.ShapeDtypeStruct((N, C), jnp.float32),
    )(x, bi, chn, dinv, acc0, acc1, h, bg)


@jax.jit
def kernel(x, batch, edge_index, W1, b1, W2, b2, Wg, bg):
    h, chn = _run_stats(x, batch, W1, b1, W2, b2, Wg)

    acc0, acc1, dinv = _run_sc(edge_index, h)

    return _run_combine(x, batch, chn, dinv, acc0, acc1, h, bg)


# speedup vs baseline: 131.5848x; 1.2106x over previous
import jax
import jax.numpy as jnp
from jax import lax
from jax.experimental import pallas as pl
from jax.experimental.pallas import tpu as pltpu
from jax.experimental.pallas import tpu_sc as plsc

NC, NS, LANES = 2, 16, 16
NW = NC * NS

N, C, E, G = 10000, 128, 320000, 64
BN = 2048
GRID = (N + BN - 1) // BN

COLS = 1024
NPAD = NS * COLS
PADNODE = NPAD - 1

EW = E // NW
CH_W = EW // 16
ROWS_W = CH_W // 8 + 1
BUF_W = (EW // 128 + 2) * 128
FIRE = 16


def _h_body(x_ref, wg_ref, h_ref):
    i = pl.program_id(0)
    hv = jnp.dot(x_ref[...], wg_ref[...], preferred_element_type=jnp.float32)
    h_ref[pl.ds(i * BN, BN)] = hv.reshape(BN)


def _run_h(x, Wg):
    return pl.pallas_call(
        _h_body,
        grid=(GRID,),
        in_specs=[
            pl.BlockSpec((BN, C), lambda i: (i, 0)),
            pl.BlockSpec((C, 1), lambda i: (0, 0)),
        ],
        out_specs=pl.BlockSpec((NPAD,), lambda i: (0,)),
        out_shape=jax.ShapeDtypeStruct((NPAD,), jnp.float32),
    )(x, Wg)


def _stats_body(x_ref, b_ref, w1_ref, b1_ref, w2_ref, b2_ref,
                chn_ref, sums_ref, cnt_ref):
    i = pl.program_id(0)
    xb = x_ref[...]
    bi = b_ref[...].reshape(BN, 1)
    hit = bi == lax.broadcasted_iota(jnp.int32, (BN, G), 1)

    @pl.when(i == 0)
    def _():
        sums_ref[...] = jnp.zeros_like(sums_ref)
        cnt_ref[...] = jnp.zeros_like(cnt_ref)

    def accum(onehot, xv):
        psums = lax.dot_general(onehot, xv, (((0,), (0,)), ((), ())),
                                preferred_element_type=jnp.float32)
        pcnt = lax.dot_general(onehot, jnp.ones((BN, 1), jnp.float32),
                               (((0,), (0,)), ((), ())),
                               preferred_element_type=jnp.float32)
        sums_ref[...] += psums
        cnt_ref[...] += pcnt

    @pl.when(i < GRID - 1)
    def _():
        accum(jnp.where(hit, 1.0, 0.0), xb)

    @pl.when(i == GRID - 1)
    def _():
        rid = lax.broadcasted_iota(jnp.int32, (BN, 1), 0) + i * BN
        valid = rid < N
        accum(jnp.where(valid & hit, 1.0, 0.0), jnp.where(valid, xb, 0.0))

    @pl.when(i == GRID - 1)
    def _():
        means = sums_ref[...] / jnp.maximum(cnt_ref[...], 1.0)
        t = jnp.maximum(
            jnp.dot(means, w1_ref[...], preferred_element_type=jnp.float32)
            + b1_ref[...].reshape(1, G), 0.0)
        chn_ref[...] = jax.nn.sigmoid(
            jnp.dot(t, w2_ref[...], preferred_element_type=jnp.float32)
            + b2_ref[...].reshape(1, C))


def _run_stats(x, batch, W1, b1, W2, b2):
    return pl.pallas_call(
        _stats_body,
        grid=(GRID,),
        in_specs=[
            pl.BlockSpec((BN, C), lambda i: (i, 0)),
            pl.BlockSpec((BN,), lambda i: (i,)),
            pl.BlockSpec((C, G), lambda i: (0, 0)),
            pl.BlockSpec((G,), lambda i: (0,)),
            pl.BlockSpec((G, C), lambda i: (0, 0)),
            pl.BlockSpec((C,), lambda i: (0,)),
        ],
        out_specs=pl.BlockSpec((G, C), lambda i: (0, 0)),
        out_shape=jax.ShapeDtypeStruct((G, C), jnp.float32),
        scratch_shapes=[
            pltpu.VMEM((G, C), jnp.float32),
            pltpu.VMEM((G, 1), jnp.float32),
        ],
    )(x, batch, W1, b1, W2, b2)


def _rsqrt3(d):
    yi = plsc.bitcast(d, jnp.int32)
    yi = 0x5F3759DF - lax.shift_right_arithmetic(yi, 1)
    y = plsc.bitcast(yi, jnp.float32)
    half = 0.5 * d
    y = y * (1.5 - half * y * y)
    y = y * (1.5 - half * y * y)
    y = y * (1.5 - half * y * y)
    return y


def _fire_scatter(n_rows, vals_row, idx2d, target, sem):
    for base in range(0, n_rows, FIRE):
        k = min(FIRE, n_rows - base)
        descs = [
            pltpu.async_copy(vals_row(base + j),
                             target.at[idx2d.at[base + j]], sem, add=True)
            for j in range(k)
        ]
        for dsc in descs:
            dsc.wait()


def _edge_span(wid):
    pre = (wid * EW) % 128
    ab = pl.multiple_of(wid * EW - pre, 128)
    return pre, ab


def _start_edge_load(ei, ab, wid, eb_v, sem):
    @pl.when(wid < NW - 1)
    def _():
        pltpu.async_copy(ei.at[:, pl.ds(ab, BUF_W)], eb_v, sem)

    @pl.when(wid == NW - 1)
    def _():
        pltpu.async_copy(ei.at[:, pl.ds(ab, BUF_W - 128)],
                         eb_v.at[:, pl.ds(0, BUF_W - 128)], sem)


def _wait_edge_load(ei, ab, wid, eb_v, sem):
    @pl.when(wid < NW - 1)
    def _():
        pltpu.make_async_copy(ei.at[:, pl.ds(ab, BUF_W)], eb_v, sem).wait()

    @pl.when(wid == NW - 1)
    def _():
        pltpu.make_async_copy(ei.at[:, pl.ds(ab, BUF_W - 128)],
                              eb_v.at[:, pl.ds(0, BUF_W - 128)], sem).wait()


def _fill_zeros(zeros_v):
    def zbody(j, _):
        zeros_v[pl.ds(j * 16, 16)] = jnp.zeros((16,), jnp.float32)
        return 0

    lax.fori_loop(0, COLS // 16, zbody, 0)


def _deg_body(ei, degp, zeros_v, ones_v, eb_v, dst2_v, deg_s, sem, semw):
    c = lax.axis_index("c")
    s = lax.axis_index("s")
    wid = c * NS + s
    padv = jnp.full((16,), PADNODE, jnp.int32)
    pre, ab = _edge_span(wid)

    _start_edge_load(ei, ab, wid, eb_v, semw)
    _fill_zeros(zeros_v)
    for k in range(128 // 16):
        ones_v[pl.ds(k * 16, 16)] = jnp.ones((16,), jnp.float32)
    pltpu.sync_copy(zeros_v, deg_s.at[pl.ds(s * COLS, COLS)])
    plsc.subcore_barrier()

    _wait_edge_load(ei, ab, wid, eb_v, semw)

    def cbody(j, _):
        off = jnp.minimum(pre + j * 16, BUF_W - 16)
        v = jnp.where(j < CH_W, eb_v[1, pl.ds(off, 16)], padv)
        dst2_v[j >> 3, pl.ds((j & 7) * 16, 16)] = v
        return 0

    lax.fori_loop(0, ROWS_W * 8, cbody, 0)
    _fire_scatter(ROWS_W, lambda j: ones_v, dst2_v, deg_s, sem)
    plsc.subcore_barrier()

    pltpu.sync_copy(deg_s.at[pl.ds(s * COLS, COLS)],
                    degp.at[c, pl.ds(s * COLS, COLS)])


def _run_deg(ei):
    mesh = plsc.VectorSubcoreMesh(core_axis_name="c", subcore_axis_name="s",
                                  num_cores=NC, num_subcores=NS)
    return pl.kernel(
        _deg_body,
        out_type=jax.ShapeDtypeStruct((NC, NPAD), jnp.float32),
        mesh=mesh,
        compiler_params=pltpu.CompilerParams(needs_layout_passes=False),
        scratch_types=[
            pltpu.VMEM((COLS,), jnp.float32),
            pltpu.VMEM((128,), jnp.float32),
            pltpu.VMEM((2, BUF_W), jnp.int32),
            pltpu.VMEM((ROWS_W, 128), jnp.int32),
            pltpu.VMEM_SHARED((NPAD,), jnp.float32),
            pltpu.SemaphoreType.DMA,
            pltpu.SemaphoreType.DMA,
        ],
    )(ei)


def _main_body(ei, h1d, degp, acc0_out, acc1_out, dinv_out,
               zeros_v, eb_v, dst2_v, qv_v, q_v, d_v, d2_v, h_v,
               q_s, acc_s, sem, semw, semh):
    c = lax.axis_index("c")
    s = lax.axis_index("s")
    wid = c * NS + s
    padv = jnp.full((16,), PADNODE, jnp.int32)
    pre, ab = _edge_span(wid)

    _start_edge_load(ei, ab, wid, eb_v, semw)
    pltpu.async_copy(h1d.at[pl.ds(s * COLS, COLS)], h_v, semh)
    _fill_zeros(zeros_v)
    pltpu.sync_copy(zeros_v, acc_s.at[pl.ds(s * COLS, COLS)])
    pltpu.sync_copy(degp.at[0, pl.ds(s * COLS, COLS)], d_v)
    pltpu.sync_copy(degp.at[1, pl.ds(s * COLS, COLS)], d2_v)
    plsc.subcore_barrier()

    pltpu.make_async_copy(h1d.at[pl.ds(0, COLS)], h_v, semh).wait()

    def qbody(j, _):
        sl = pl.ds(j * 16, 16)
        y = _rsqrt3(d_v[sl] + d2_v[sl] + 1.0)
        d_v[sl] = y
        h_v[sl] = y * h_v[sl]
        return 0

    lax.fori_loop(0, COLS // 16, qbody, 0)
    pltpu.sync_copy(h_v, q_s.at[pl.ds(s * COLS, COLS)])

    @pl.when(c == 0)
    def _():
        pltpu.sync_copy(d_v, dinv_out.at[pl.ds(s * COLS, COLS)])

    plsc.subcore_barrier()

    pltpu.sync_copy(q_s, q_v)
    _wait_edge_load(ei, ab, wid, eb_v, semw)

    def gbody(j, _):
        off = jnp.minimum(pre + j * 16, BUF_W - 16)
        sl16 = pl.ds((j & 7) * 16, 16)
        row = j >> 3
        sidx = jnp.where(j < CH_W, eb_v[0, pl.ds(off, 16)], padv)
        didx = jnp.where(j < CH_W, eb_v[1, pl.ds(off, 16)], padv)
        qv_v[row, sl16] = plsc.load_gather(q_v, [sidx])
        dst2_v[row, sl16] = didx
        return 0

    lax.fori_loop(0, ROWS_W * 8, gbody, 0)
    _fire_scatter(ROWS_W, lambda j: qv_v.at[j], dst2_v, acc_s, sem)
    plsc.subcore_barrier()

    @pl.when(c == 0)
    def _():
        pltpu.sync_copy(acc_s.at[pl.ds(s * COLS, COLS)],
                        acc0_out.at[pl.ds(s * COLS, COLS)])

    @pl.when(c == 1)
    def _():
        pltpu.sync_copy(acc_s.at[pl.ds(s * COLS, COLS)],
                        acc1_out.at[pl.ds(s * COLS, COLS)])


def _run_main(ei, h1d, degp):
    mesh = plsc.VectorSubcoreMesh(core_axis_name="c", subcore_axis_name="s",
                                  num_cores=NC, num_subcores=NS)
    return pl.kernel(
        _main_body,
        out_type=(
            jax.ShapeDtypeStruct((NPAD,), jnp.float32),
            jax.ShapeDtypeStruct((NPAD,), jnp.float32),
            jax.ShapeDtypeStruct((NPAD,), jnp.float32),
        ),
        mesh=mesh,
        compiler_params=pltpu.CompilerParams(needs_layout_passes=False),
        scratch_types=[
            pltpu.VMEM((COLS,), jnp.float32),
            pltpu.VMEM((2, BUF_W), jnp.int32),
            pltpu.VMEM((ROWS_W, 128), jnp.int32),
            pltpu.VMEM((ROWS_W, 128), jnp.float32),
            pltpu.VMEM((NPAD,), jnp.float32),
            pltpu.VMEM((COLS,), jnp.float32),
            pltpu.VMEM((COLS,), jnp.float32),
            pltpu.VMEM((COLS,), jnp.float32),
            pltpu.VMEM_SHARED((NPAD,), jnp.float32),
            pltpu.VMEM_SHARED((NPAD,), jnp.float32),
            pltpu.SemaphoreType.DMA,
            pltpu.SemaphoreType.DMA,
            pltpu.SemaphoreType.DMA,
        ],
    )(ei, h1d, degp)


def _combine_body(x_ref, b_ref, chn_ref, dinv_ref, acc0_ref, acc1_ref,
                  h1_ref, bg_ref, o_ref):
    i = pl.program_id(0)
    xb = x_ref[...]
    bi = b_ref[...].reshape(BN, 1)
    onehot = (bi == lax.broadcasted_iota(jnp.int32, (BN, G), 1)
              ).astype(jnp.float32)
    chn_rows = jnp.dot(onehot, chn_ref[...], preferred_element_type=jnp.float32)
    sl = pl.ds(i * BN, BN)
    dinv = dinv_ref[sl]
    gcn = dinv * (acc0_ref[sl] + acc1_ref[sl]) + dinv * dinv * h1_ref[sl]
    spa = jax.nn.sigmoid(gcn + bg_ref[...]).reshape(BN, 1)
    o_ref[...] = xb * (1.0 + spa + chn_rows)


def _run_combine(x, batch, chn, dinv, acc0, acc1, h, bg):
    full = pl.BlockSpec((NPAD,), lambda i: (0,))
    return pl.pallas_call(
        _combine_body,
        grid=(GRID,),
        in_specs=[
            pl.BlockSpec((BN, C), lambda i: (i, 0)),
            pl.BlockSpec((BN,), lambda i: (i,)),
            pl.BlockSpec((G, C), lambda i: (0, 0)),
            full, full, full, full,
            pl.BlockSpec((1,), lambda i: (0,)),
        ],
        out_specs=pl.BlockSpec((BN, C), lambda i: (i, 0)),
        out_shape=jax.ShapeDtypeStruct((N, C), jnp.float32),
    )(x, batch, chn, dinv, acc0, acc1, h, bg)


@jax.jit
def kernel(x, batch, edge_index, W1, b1, W2, b2, Wg, bg):
    h = _run_h(x, Wg)
    degp = _run_deg(edge_index)
    chn = _run_stats(x, batch, W1, b1, W2, b2)
    acc0, acc1, dinv = _run_main(edge_index, h, degp)
    return _run_combine(x, batch, chn, dinv, acc0, acc1, h, bg)


# speedup vs baseline: 144.7511x; 1.1001x over previous
import jax
import jax.numpy as jnp
from jax import lax
from jax.experimental import pallas as pl
from jax.experimental.pallas import tpu as pltpu
from jax.experimental.pallas import tpu_sc as plsc

NC, NS, LANES = 2, 16, 16
NW = NC * NS

N, C, E, G = 10000, 128, 320000, 64
BN = 2048
GRID = (N + BN - 1) // BN

COLS = 1024
NPAD = NS * COLS
PADNODE = NPAD - 1

EW = E // NW
CH_W = EW // 16
ROWS_W = CH_W // 8 + 1
BUF_W = (EW // 128 + 2) * 128
FIRE = 16


def _h_body(x_ref, wg_ref, h_ref):
    i = pl.program_id(0)
    hv = jnp.dot(x_ref[...], wg_ref[...], preferred_element_type=jnp.float32)
    h_ref[pl.ds(i * BN, BN)] = hv.reshape(BN)


def _run_h(x, Wg):
    return pl.pallas_call(
        _h_body,
        grid=(GRID,),
        in_specs=[
            pl.BlockSpec((BN, C), lambda i: (i, 0)),
            pl.BlockSpec((C, 1), lambda i: (0, 0)),
        ],
        out_specs=pl.BlockSpec((NPAD,), lambda i: (0,)),
        out_shape=jax.ShapeDtypeStruct((NPAD,), jnp.float32),
    )(x, Wg)


def _stats_body(x_ref, b_ref, w1_ref, b1_ref, w2_ref, b2_ref,
                chn_ref, sums_ref, cnt_ref):
    i = pl.program_id(0)
    xb = x_ref[...]
    bi = b_ref[...].reshape(BN, 1)
    hit = bi == lax.broadcasted_iota(jnp.int32, (BN, G), 1)

    @pl.when(i == 0)
    def _():
        sums_ref[...] = jnp.zeros_like(sums_ref)
        cnt_ref[...] = jnp.zeros_like(cnt_ref)

    def accum(onehot, xv):
        psums = lax.dot_general(onehot, xv, (((0,), (0,)), ((), ())),
                                preferred_element_type=jnp.float32)
        pcnt = lax.dot_general(onehot, jnp.ones((BN, 1), jnp.float32),
                               (((0,), (0,)), ((), ())),
                               preferred_element_type=jnp.float32)
        sums_ref[...] += psums
        cnt_ref[...] += pcnt

    @pl.when(i < GRID - 1)
    def _():
        accum(jnp.where(hit, 1.0, 0.0), xb)

    @pl.when(i == GRID - 1)
    def _():
        rid = lax.broadcasted_iota(jnp.int32, (BN, 1), 0) + i * BN
        valid = rid < N
        accum(jnp.where(valid & hit, 1.0, 0.0), jnp.where(valid, xb, 0.0))

    @pl.when(i == GRID - 1)
    def _():
        means = sums_ref[...] / jnp.maximum(cnt_ref[...], 1.0)
        t = jnp.maximum(
            jnp.dot(means, w1_ref[...], preferred_element_type=jnp.float32)
            + b1_ref[...].reshape(1, G), 0.0)
        chn_ref[...] = jax.nn.sigmoid(
            jnp.dot(t, w2_ref[...], preferred_element_type=jnp.float32)
            + b2_ref[...].reshape(1, C))


def _run_stats(x, batch, W1, b1, W2, b2):
    return pl.pallas_call(
        _stats_body,
        grid=(GRID,),
        in_specs=[
            pl.BlockSpec((BN, C), lambda i: (i, 0)),
            pl.BlockSpec((BN,), lambda i: (i,)),
            pl.BlockSpec((C, G), lambda i: (0, 0)),
            pl.BlockSpec((G,), lambda i: (0,)),
            pl.BlockSpec((G, C), lambda i: (0, 0)),
            pl.BlockSpec((C,), lambda i: (0,)),
        ],
        out_specs=pl.BlockSpec((G, C), lambda i: (0, 0)),
        out_shape=jax.ShapeDtypeStruct((G, C), jnp.float32),
        scratch_shapes=[
            pltpu.VMEM((G, C), jnp.float32),
            pltpu.VMEM((G, 1), jnp.float32),
        ],
    )(x, batch, W1, b1, W2, b2)


def _rsqrt3(d):
    yi = plsc.bitcast(d, jnp.int32)
    yi = 0x5F3759DF - lax.shift_right_arithmetic(yi, 1)
    y = plsc.bitcast(yi, jnp.float32)
    half = 0.5 * d
    y = y * (1.5 - half * y * y)
    y = y * (1.5 - half * y * y)
    y = y * (1.5 - half * y * y)
    return y


def _pipe_scatter(vals_row, idx2d, target, sem, build_chunk):
    prev = []
    for base in range(0, ROWS_W, FIRE):
        k = min(FIRE, ROWS_W - base)
        build_chunk(base, k)
        cur = [
            pltpu.async_copy(vals_row(base + j),
                             target.at[idx2d.at[base + j]], sem, add=True)
            for j in range(k)
        ]
        for dsc in prev:
            dsc.wait()
        prev = cur
    for dsc in prev:
        dsc.wait()


def _edge_span(wid):
    pre = (wid * EW) % 128
    ab = pl.multiple_of(wid * EW - pre, 128)
    return pre, ab


def _start_edge_load(ei, ab, wid, eb_v, sem):
    @pl.when(wid < NW - 1)
    def _():
        pltpu.async_copy(ei.at[:, pl.ds(ab, BUF_W)], eb_v, sem)

    @pl.when(wid == NW - 1)
    def _():
        pltpu.async_copy(ei.at[:, pl.ds(ab, BUF_W - 128)],
                         eb_v.at[:, pl.ds(0, BUF_W - 128)], sem)


def _wait_edge_load(ei, ab, wid, eb_v, sem):
    @pl.when(wid < NW - 1)
    def _():
        pltpu.make_async_copy(ei.at[:, pl.ds(ab, BUF_W)], eb_v, sem).wait()

    @pl.when(wid == NW - 1)
    def _():
        pltpu.make_async_copy(ei.at[:, pl.ds(ab, BUF_W - 128)],
                              eb_v.at[:, pl.ds(0, BUF_W - 128)], sem).wait()


def _fill_zeros(zeros_v):
    def zbody(j, _):
        zeros_v[pl.ds(j * 16, 16)] = jnp.zeros((16,), jnp.float32)
        return 0

    lax.fori_loop(0, COLS // 16, zbody, 0)


def _deg_body(ei, degp, zeros_v, ones_v, eb_v, dst2_v, deg_s, sem, semw):
    c = lax.axis_index("c")
    s = lax.axis_index("s")
    wid = c * NS + s
    padv = jnp.full((16,), PADNODE, jnp.int32)
    pre, ab = _edge_span(wid)

    _start_edge_load(ei, ab, wid, eb_v, semw)
    _fill_zeros(zeros_v)
    for k in range(128 // 16):
        ones_v[pl.ds(k * 16, 16)] = jnp.ones((16,), jnp.float32)
    pltpu.sync_copy(zeros_v, deg_s.at[pl.ds(s * COLS, COLS)])
    plsc.subcore_barrier()

    _wait_edge_load(ei, ab, wid, eb_v, semw)

    def cbody(j, _):
        off = jnp.minimum(pre + j * 16, BUF_W - 16)
        v = jnp.where(j < CH_W, eb_v[1, pl.ds(off, 16)], padv)
        dst2_v[j >> 3, pl.ds((j & 7) * 16, 16)] = v
        return 0

    def build(base, k):
        lax.fori_loop(base * 8, (base + k) * 8, cbody, 0)

    _pipe_scatter(lambda j: ones_v, dst2_v, deg_s, sem, build)
    plsc.subcore_barrier()

    pltpu.sync_copy(deg_s.at[pl.ds(s * COLS, COLS)],
                    degp.at[c, pl.ds(s * COLS, COLS)])


def _run_deg(ei):
    mesh = plsc.VectorSubcoreMesh(core_axis_name="c", subcore_axis_name="s",
                                  num_cores=NC, num_subcores=NS)
    return pl.kernel(
        _deg_body,
        out_type=jax.ShapeDtypeStruct((NC, NPAD), jnp.float32),
        mesh=mesh,
        compiler_params=pltpu.CompilerParams(needs_layout_passes=False),
        scratch_types=[
            pltpu.VMEM((COLS,), jnp.float32),
            pltpu.VMEM((128,), jnp.float32),
            pltpu.VMEM((2, BUF_W), jnp.int32),
            pltpu.VMEM((ROWS_W, 128), jnp.int32),
            pltpu.VMEM_SHARED((NPAD,), jnp.float32),
            pltpu.SemaphoreType.DMA,
            pltpu.SemaphoreType.DMA,
        ],
    )(ei)


def _main_body(ei, h1d, degp, acc0_out, acc1_out, dinv_out,
               zeros_v, eb_v, dst2_v, qv_v, q_v, d_v, d2_v, h_v,
               q_s, acc_s, sem, semw, semh):
    c = lax.axis_index("c")
    s = lax.axis_index("s")
    wid = c * NS + s
    padv = jnp.full((16,), PADNODE, jnp.int32)
    pre, ab = _edge_span(wid)

    _start_edge_load(ei, ab, wid, eb_v, semw)
    pltpu.async_copy(h1d.at[pl.ds(s * COLS, COLS)], h_v, semh)
    _fill_zeros(zeros_v)
    pltpu.sync_copy(zeros_v, acc_s.at[pl.ds(s * COLS, COLS)])
    pltpu.sync_copy(degp.at[0, pl.ds(s * COLS, COLS)], d_v)
    pltpu.sync_copy(degp.at[1, pl.ds(s * COLS, COLS)], d2_v)
    plsc.subcore_barrier()

    pltpu.make_async_copy(h1d.at[pl.ds(0, COLS)], h_v, semh).wait()

    def qbody(j, _):
        sl = pl.ds(j * 16, 16)
        y = _rsqrt3(d_v[sl] + d2_v[sl] + 1.0)
        d_v[sl] = y
        h_v[sl] = y * h_v[sl]
        return 0

    lax.fori_loop(0, COLS // 16, qbody, 0)
    pltpu.sync_copy(h_v, q_s.at[pl.ds(s * COLS, COLS)])

    @pl.when(c == 0)
    def _():
        pltpu.sync_copy(d_v, dinv_out.at[pl.ds(s * COLS, COLS)])

    plsc.subcore_barrier()

    pltpu.sync_copy(q_s, q_v)
    _wait_edge_load(ei, ab, wid, eb_v, semw)

    def gbody(j, _):
        off = jnp.minimum(pre + j * 16, BUF_W - 16)
        sl16 = pl.ds((j & 7) * 16, 16)
        row = j >> 3
        sidx = jnp.where(j < CH_W, eb_v[0, pl.ds(off, 16)], padv)
        didx = jnp.where(j < CH_W, eb_v[1, pl.ds(off, 16)], padv)
        qv_v[row, sl16] = plsc.load_gather(q_v, [sidx])
        dst2_v[row, sl16] = didx
        return 0

    def build(base, k):
        lax.fori_loop(base * 8, (base + k) * 8, gbody, 0)

    _pipe_scatter(lambda j: qv_v.at[j], dst2_v, acc_s, sem, build)
    plsc.subcore_barrier()

    @pl.when(c == 0)
    def _():
        pltpu.sync_copy(acc_s.at[pl.ds(s * COLS, COLS)],
                        acc0_out.at[pl.ds(s * COLS, COLS)])

    @pl.when(c == 1)
    def _():
        pltpu.sync_copy(acc_s.at[pl.ds(s * COLS, COLS)],
                        acc1_out.at[pl.ds(s * COLS, COLS)])


def _run_main(ei, h1d, degp):
    mesh = plsc.VectorSubcoreMesh(core_axis_name="c", subcore_axis_name="s",
                                  num_cores=NC, num_subcores=NS)
    return pl.kernel(
        _main_body,
        out_type=(
            jax.ShapeDtypeStruct((NPAD,), jnp.float32),
            jax.ShapeDtypeStruct((NPAD,), jnp.float32),
            jax.ShapeDtypeStruct((NPAD,), jnp.float32),
        ),
        mesh=mesh,
        compiler_params=pltpu.CompilerParams(needs_layout_passes=False),
        scratch_types=[
            pltpu.VMEM((COLS,), jnp.float32),
            pltpu.VMEM((2, BUF_W), jnp.int32),
            pltpu.VMEM((ROWS_W, 128), jnp.int32),
            pltpu.VMEM((ROWS_W, 128), jnp.float32),
            pltpu.VMEM((NPAD,), jnp.float32),
            pltpu.VMEM((COLS,), jnp.float32),
            pltpu.VMEM((COLS,), jnp.float32),
            pltpu.VMEM((COLS,), jnp.float32),
            pltpu.VMEM_SHARED((NPAD,), jnp.float32),
            pltpu.VMEM_SHARED((NPAD,), jnp.float32),
            pltpu.SemaphoreType.DMA,
            pltpu.SemaphoreType.DMA,
            pltpu.SemaphoreType.DMA,
        ],
    )(ei, h1d, degp)


def _combine_body(x_ref, b_ref, chn_ref, dinv_ref, acc0_ref, acc1_ref,
                  h1_ref, bg_ref, o_ref):
    i = pl.program_id(0)
    xb = x_ref[...]
    bi = b_ref[...].reshape(BN, 1)
    onehot = (bi == lax.broadcasted_iota(jnp.int32, (BN, G), 1)
              ).astype(jnp.float32)
    chn_rows = jnp.dot(onehot, chn_ref[...], preferred_element_type=jnp.float32)
    sl = pl.ds(i * BN, BN)
    dinv = dinv_ref[sl]
    gcn = dinv * (acc0_ref[sl] + acc1_ref[sl]) + dinv * dinv * h1_ref[sl]
    spa = jax.nn.sigmoid(gcn + bg_ref[...]).reshape(BN, 1)
    o_ref[...] = xb * (1.0 + spa + chn_rows)


def _run_combine(x, batch, chn, dinv, acc0, acc1, h, bg):
    full = pl.BlockSpec((NPAD,), lambda i: (0,))
    return pl.pallas_call(
        _combine_body,
        grid=(GRID,),
        in_specs=[
            pl.BlockSpec((BN, C), lambda i: (i, 0)),
            pl.BlockSpec((BN,), lambda i: (i,)),
            pl.BlockSpec((G, C), lambda i: (0, 0)),
            full, full, full, full,
            pl.BlockSpec((1,), lambda i: (0,)),
        ],
        out_specs=pl.BlockSpec((BN, C), lambda i: (i, 0)),
        out_shape=jax.ShapeDtypeStruct((N, C), jnp.float32),
    )(x, batch, chn, dinv, acc0, acc1, h, bg)


@jax.jit
def kernel(x, batch, edge_index, W1, b1, W2, b2, Wg, bg):
    h = _run_h(x, Wg)
    degp = _run_deg(edge_index)
    chn = _run_stats(x, batch, W1, b1, W2, b2)
    acc0, acc1, dinv = _run_main(edge_index, h, degp)
    return _run_combine(x, batch, chn, dinv, acc0, acc1, h, bg)


# speedup vs baseline: 160.0586x; 1.1058x over previous
import jax
import jax.numpy as jnp
from jax import lax
from jax.experimental import pallas as pl
from jax.experimental.pallas import tpu as pltpu
from jax.experimental.pallas import tpu_sc as plsc

NC, NS, LANES = 2, 16, 16
NW = NC * NS

N, C, E, G = 10000, 128, 320000, 64
BN = 2048
GRID = (N + BN - 1) // BN

COLS = 1024
NPAD = NS * COLS
PADNODE = NPAD - 1

EW = E // NW
CH_W = EW // 16
ROWS_W = CH_W // 8 + 1
BUF_W = (EW // 128 + 2) * 128
FIRE = 16


def _h_body(x_ref, wg_ref, h_ref):
    i = pl.program_id(0)
    hv = jnp.dot(x_ref[...], wg_ref[...], preferred_element_type=jnp.float32)
    h_ref[pl.ds(i * BN, BN)] = hv.reshape(BN)


def _run_h(x, Wg):
    return pl.pallas_call(
        _h_body,
        grid=(GRID,),
        in_specs=[
            pl.BlockSpec((BN, C), lambda i: (i, 0)),
            pl.BlockSpec((C, 1), lambda i: (0, 0)),
        ],
        out_specs=pl.BlockSpec((NPAD,), lambda i: (0,)),
        out_shape=jax.ShapeDtypeStruct((NPAD,), jnp.float32),
    )(x, Wg)


def _stats_body(x_ref, b_ref, w1_ref, b1_ref, w2_ref, b2_ref,
                chn_ref, sums_ref, cnt_ref):
    i = pl.program_id(0)
    xb = x_ref[...]
    bi = b_ref[...].reshape(BN, 1)
    hit = bi == lax.broadcasted_iota(jnp.int32, (BN, G), 1)

    @pl.when(i == 0)
    def _():
        sums_ref[...] = jnp.zeros_like(sums_ref)
        cnt_ref[...] = jnp.zeros_like(cnt_ref)

    def accum(onehot, xv):
        psums = lax.dot_general(onehot, xv, (((0,), (0,)), ((), ())),
                                preferred_element_type=jnp.float32)
        pcnt = lax.dot_general(onehot, jnp.ones((BN, 1), jnp.float32),
                               (((0,), (0,)), ((), ())),
                               preferred_element_type=jnp.float32)
        sums_ref[...] += psums
        cnt_ref[...] += pcnt

    @pl.when(i < GRID - 1)
    def _():
        accum(jnp.where(hit, 1.0, 0.0), xb)

    @pl.when(i == GRID - 1)
    def _():
        rid = lax.broadcasted_iota(jnp.int32, (BN, 1), 0) + i * BN
        valid = rid < N
        accum(jnp.where(valid & hit, 1.0, 0.0), jnp.where(valid, xb, 0.0))

    @pl.when(i == GRID - 1)
    def _():
        means = sums_ref[...] / jnp.maximum(cnt_ref[...], 1.0)
        t = jnp.maximum(
            jnp.dot(means, w1_ref[...], preferred_element_type=jnp.float32)
            + b1_ref[...].reshape(1, G), 0.0)
        chn_ref[...] = jax.nn.sigmoid(
            jnp.dot(t, w2_ref[...], preferred_element_type=jnp.float32)
            + b2_ref[...].reshape(1, C))


def _run_stats(x, batch, W1, b1, W2, b2):
    return pl.pallas_call(
        _stats_body,
        grid=(GRID,),
        in_specs=[
            pl.BlockSpec((BN, C), lambda i: (i, 0)),
            pl.BlockSpec((BN,), lambda i: (i,)),
            pl.BlockSpec((C, G), lambda i: (0, 0)),
            pl.BlockSpec((G,), lambda i: (0,)),
            pl.BlockSpec((G, C), lambda i: (0, 0)),
            pl.BlockSpec((C,), lambda i: (0,)),
        ],
        out_specs=pl.BlockSpec((G, C), lambda i: (0, 0)),
        out_shape=jax.ShapeDtypeStruct((G, C), jnp.float32),
        scratch_shapes=[
            pltpu.VMEM((G, C), jnp.float32),
            pltpu.VMEM((G, 1), jnp.float32),
        ],
    )(x, batch, W1, b1, W2, b2)


def _rsqrt3(d):
    yi = plsc.bitcast(d, jnp.int32)
    yi = 0x5F3759DF - lax.shift_right_arithmetic(yi, 1)
    y = plsc.bitcast(yi, jnp.float32)
    half = 0.5 * d
    y = y * (1.5 - half * y * y)
    y = y * (1.5 - half * y * y)
    y = y * (1.5 - half * y * y)
    return y


def _pipe_scatter(vals_row, idx2d, target, sem, build_chunk):
    prev = []
    for base in range(0, ROWS_W, FIRE):
        k = min(FIRE, ROWS_W - base)
        build_chunk(base, k)
        cur = [
            pltpu.async_copy(vals_row(base + j),
                             target.at[idx2d.at[base + j]], sem, add=True)
            for j in range(k)
        ]
        for dsc in prev:
            dsc.wait()
        prev = cur
    for dsc in prev:
        dsc.wait()


def _edge_span(wid):
    pre = (wid * EW) % 128
    ab = pl.multiple_of(wid * EW - pre, 128)
    return pre, ab


def _start_edge_load(ei, ab, wid, eb_v, sem):
    @pl.when(wid < NW - 1)
    def _():
        pltpu.async_copy(ei.at[:, pl.ds(ab, BUF_W)], eb_v, sem)

    @pl.when(wid == NW - 1)
    def _():
        pltpu.async_copy(ei.at[:, pl.ds(ab, BUF_W - 128)],
                         eb_v.at[:, pl.ds(0, BUF_W - 128)], sem)


def _wait_edge_load(ei, ab, wid, eb_v, sem):
    @pl.when(wid < NW - 1)
    def _():
        pltpu.make_async_copy(ei.at[:, pl.ds(ab, BUF_W)], eb_v, sem).wait()

    @pl.when(wid == NW - 1)
    def _():
        pltpu.make_async_copy(ei.at[:, pl.ds(ab, BUF_W - 128)],
                              eb_v.at[:, pl.ds(0, BUF_W - 128)], sem).wait()


def _fill_zeros(zeros_v):
    @plsc.parallel_loop(0, COLS // 16, unroll=4)
    def _(j):
        zeros_v[pl.ds(j * 16, 16)] = jnp.zeros((16,), jnp.float32)


def _deg_body(ei, degp, zeros_v, ones_v, eb_v, dst2_v, deg_s, sem, semw):
    c = lax.axis_index("c")
    s = lax.axis_index("s")
    wid = c * NS + s
    padv = jnp.full((16,), PADNODE, jnp.int32)
    pre, ab = _edge_span(wid)

    _start_edge_load(ei, ab, wid, eb_v, semw)
    _fill_zeros(zeros_v)
    for k in range(128 // 16):
        ones_v[pl.ds(k * 16, 16)] = jnp.ones((16,), jnp.float32)
    pltpu.sync_copy(zeros_v, deg_s.at[pl.ds(s * COLS, COLS)])
    plsc.subcore_barrier()

    _wait_edge_load(ei, ab, wid, eb_v, semw)

    def build(base, k):
        @plsc.parallel_loop(base * 8, (base + k) * 8, unroll=4)
        def _(j):
            off = jnp.minimum(pre + j * 16, BUF_W - 16)
            v = jnp.where(j < CH_W, eb_v[1, pl.ds(off, 16)], padv)
            dst2_v[j >> 3, pl.ds((j & 7) * 16, 16)] = v

    _pipe_scatter(lambda j: ones_v, dst2_v, deg_s, sem, build)
    plsc.subcore_barrier()

    pltpu.sync_copy(deg_s.at[pl.ds(s * COLS, COLS)],
                    degp.at[c, pl.ds(s * COLS, COLS)])


def _run_deg(ei):
    mesh = plsc.VectorSubcoreMesh(core_axis_name="c", subcore_axis_name="s",
                                  num_cores=NC, num_subcores=NS)
    return pl.kernel(
        _deg_body,
        out_type=jax.ShapeDtypeStruct((NC, NPAD), jnp.float32),
        mesh=mesh,
        compiler_params=pltpu.CompilerParams(needs_layout_passes=False),
        scratch_types=[
            pltpu.VMEM((COLS,), jnp.float32),
            pltpu.VMEM((128,), jnp.float32),
            pltpu.VMEM((2, BUF_W), jnp.int32),
            pltpu.VMEM((ROWS_W, 128), jnp.int32),
            pltpu.VMEM_SHARED((NPAD,), jnp.float32),
            pltpu.SemaphoreType.DMA,
            pltpu.SemaphoreType.DMA,
        ],
    )(ei)


def _main_body(ei, h1d, degp, acc0_out, acc1_out, dinv_out,
               zeros_v, eb_v, dst2_v, qv_v, q_v, d_v, d2_v, h_v,
               q_s, acc_s, sem, semw, semh):
    c = lax.axis_index("c")
    s = lax.axis_index("s")
    wid = c * NS + s
    padv = jnp.full((16,), PADNODE, jnp.int32)
    pre, ab = _edge_span(wid)

    _start_edge_load(ei, ab, wid, eb_v, semw)
    pltpu.async_copy(h1d.at[pl.ds(s * COLS, COLS)], h_v, semh)
    _fill_zeros(zeros_v)
    pltpu.sync_copy(zeros_v, acc_s.at[pl.ds(s * COLS, COLS)])
    pltpu.sync_copy(degp.at[0, pl.ds(s * COLS, COLS)], d_v)
    pltpu.sync_copy(degp.at[1, pl.ds(s * COLS, COLS)], d2_v)
    plsc.subcore_barrier()

    pltpu.make_async_copy(h1d.at[pl.ds(0, COLS)], h_v, semh).wait()

    @plsc.parallel_loop(0, COLS // 16, unroll=4)
    def _(j):
        sl = pl.ds(j * 16, 16)
        y = _rsqrt3(d_v[sl] + d2_v[sl] + 1.0)
        d_v[sl] = y
        h_v[sl] = y * h_v[sl]
    pltpu.sync_copy(h_v, q_s.at[pl.ds(s * COLS, COLS)])

    @pl.when(c == 0)
    def _():
        pltpu.sync_copy(d_v, dinv_out.at[pl.ds(s * COLS, COLS)])

    plsc.subcore_barrier()

    pltpu.sync_copy(q_s, q_v)
    _wait_edge_load(ei, ab, wid, eb_v, semw)

    def build(base, k):
        @plsc.parallel_loop(base * 8, (base + k) * 8, unroll=4)
        def _(j):
            off = jnp.minimum(pre + j * 16, BUF_W - 16)
            sl16 = pl.ds((j & 7) * 16, 16)
            row = j >> 3
            sidx = jnp.where(j < CH_W, eb_v[0, pl.ds(off, 16)], padv)
            didx = jnp.where(j < CH_W, eb_v[1, pl.ds(off, 16)], padv)
            qv_v[row, sl16] = plsc.load_gather(q_v, [sidx])
            dst2_v[row, sl16] = didx

    _pipe_scatter(lambda j: qv_v.at[j], dst2_v, acc_s, sem, build)
    plsc.subcore_barrier()

    @pl.when(c == 0)
    def _():
        pltpu.sync_copy(acc_s.at[pl.ds(s * COLS, COLS)],
                        acc0_out.at[pl.ds(s * COLS, COLS)])

    @pl.when(c == 1)
    def _():
        pltpu.sync_copy(acc_s.at[pl.ds(s * COLS, COLS)],
                        acc1_out.at[pl.ds(s * COLS, COLS)])


def _run_main(ei, h1d, degp):
    mesh = plsc.VectorSubcoreMesh(core_axis_name="c", subcore_axis_name="s",
                                  num_cores=NC, num_subcores=NS)
    return pl.kernel(
        _main_body,
        out_type=(
            jax.ShapeDtypeStruct((NPAD,), jnp.float32),
            jax.ShapeDtypeStruct((NPAD,), jnp.float32),
            jax.ShapeDtypeStruct((NPAD,), jnp.float32),
        ),
        mesh=mesh,
        compiler_params=pltpu.CompilerParams(needs_layout_passes=False),
        scratch_types=[
            pltpu.VMEM((COLS,), jnp.float32),
            pltpu.VMEM((2, BUF_W), jnp.int32),
            pltpu.VMEM((ROWS_W, 128), jnp.int32),
            pltpu.VMEM((ROWS_W, 128), jnp.float32),
            pltpu.VMEM((NPAD,), jnp.float32),
            pltpu.VMEM((COLS,), jnp.float32),
            pltpu.VMEM((COLS,), jnp.float32),
            pltpu.VMEM((COLS,), jnp.float32),
            pltpu.VMEM_SHARED((NPAD,), jnp.float32),
            pltpu.VMEM_SHARED((NPAD,), jnp.float32),
            pltpu.SemaphoreType.DMA,
            pltpu.SemaphoreType.DMA,
            pltpu.SemaphoreType.DMA,
        ],
    )(ei, h1d, degp)


def _combine_body(x_ref, b_ref, chn_ref, dinv_ref, acc0_ref, acc1_ref,
                  h1_ref, bg_ref, o_ref):
    i = pl.program_id(0)
    xb = x_ref[...]
    bi = b_ref[...].reshape(BN, 1)
    onehot = (bi == lax.broadcasted_iota(jnp.int32, (BN, G), 1)
              ).astype(jnp.float32)
    chn_rows = jnp.dot(onehot, chn_ref[...], preferred_element_type=jnp.float32)
    sl = pl.ds(i * BN, BN)
    dinv = dinv_ref[sl]
    gcn = dinv * (acc0_ref[sl] + acc1_ref[sl]) + dinv * dinv * h1_ref[sl]
    spa = jax.nn.sigmoid(gcn + bg_ref[...]).reshape(BN, 1)
    o_ref[...] = xb * (1.0 + spa + chn_rows)


def _run_combine(x, batch, chn, dinv, acc0, acc1, h, bg):
    full = pl.BlockSpec((NPAD,), lambda i: (0,))
    return pl.pallas_call(
        _combine_body,
        grid=(GRID,),
        in_specs=[
            pl.BlockSpec((BN, C), lambda i: (i, 0)),
            pl.BlockSpec((BN,), lambda i: (i,)),
            pl.BlockSpec((G, C), lambda i: (0, 0)),
            full, full, full, full,
            pl.BlockSpec((1,), lambda i: (0,)),
        ],
        out_specs=pl.BlockSpec((BN, C), lambda i: (i, 0)),
        out_shape=jax.ShapeDtypeStruct((N, C), jnp.float32),
    )(x, batch, chn, dinv, acc0, acc1, h, bg)


@jax.jit
def kernel(x, batch, edge_index, W1, b1, W2, b2, Wg, bg):
    h = _run_h(x, Wg)
    degp = _run_deg(edge_index)
    chn = _run_stats(x, batch, W1, b1, W2, b2)
    acc0, acc1, dinv = _run_main(edge_index, h, degp)
    return _run_combine(x, batch, chn, dinv, acc0, acc1, h, bg)
